# baseline (speedup 1.0000x reference)
import jax
import jax.numpy as jnp
from jax import lax
from jax.experimental import pallas as pl
from jax.experimental.pallas import tpu as pltpu
from jax.experimental.pallas import tpu_sc as plsc

VOCAB = 1000
BATCH = 4096
HIST = 20

_info = plsc.get_sparse_core_info()
NC = _info.num_cores
NS = _info.num_subcores
L = _info.num_lanes
NW = NC * NS
RW = BATCH // NW
C = 32
NCHUNK = RW // C
GPC = C // L


def _mh_body(xT_hbm, out_hbm, xv, acc):
    c = lax.axis_index("c")
    s = lax.axis_index("s")
    wid = s * NC + c
    base = wid * RW

    pltpu.sync_copy(xT_hbm.at[:, pl.ds(base, RW)], xv)

    def _z(i, carry):
        acc[pl.ds(i * L, L)] = jnp.zeros((L,), jnp.float32)
        return carry

    lax.fori_loop(0, C * VOCAB // L, _z, None)

    ones = jnp.ones((L,), jnp.float32)
    zeros = jnp.zeros((L,), jnp.float32)
    lane_row = lax.iota(jnp.int32, L) * VOCAB

    for k in range(NCHUNK):
        for g in range(GPC):
            rowoff = lane_row + g * (L * VOCAB)
            for j in range(HIST):
                idx = xv[j, pl.ds(k * C + g * L, L)]
                plsc.addupdate_scatter(acc, [idx + rowoff], ones)
        pltpu.sync_copy(acc, out_hbm.at[pl.ds((base + k * C) * VOCAB, C * VOCAB)])
        if k < NCHUNK - 1:
            for g in range(GPC):
                rowoff = lane_row + g * (L * VOCAB)
                for j in range(HIST):
                    idx = xv[j, pl.ds(k * C + g * L, L)]
                    plsc.store_scatter(acc, [idx + rowoff], zeros)


def kernel(x):
    xT = x.T
    out = pl.kernel(
        _mh_body,
        out_type=jax.ShapeDtypeStruct((BATCH * VOCAB,), jnp.float32),
        mesh=plsc.VectorSubcoreMesh(core_axis_name="c", subcore_axis_name="s"),
        scratch_types=[
            pltpu.VMEM((HIST, RW), jnp.int32),
            pltpu.VMEM((C * VOCAB,), jnp.float32),
        ],
        compiler_params=pltpu.CompilerParams(needs_layout_passes=False),
    )(xT)
    return out.reshape(BATCH, VOCAB)


# speedup vs baseline: 1.1541x; 1.1541x over previous
import jax
import jax.numpy as jnp
from jax import lax
from jax.experimental import pallas as pl
from jax.experimental.pallas import tpu as pltpu
from jax.experimental.pallas import tpu_sc as plsc

VOCAB = 1000
BATCH = 4096
HIST = 20

_info = plsc.get_sparse_core_info()
NC = _info.num_cores
NS = _info.num_subcores
L = _info.num_lanes
NW = NC * NS
RW = BATCH // NW
C = 32
NCHUNK = RW // C
GPC = C // L
ZUNROLL = 16


def _mh_body(xT_hbm, out_hbm, xv, acc0, acc1, sem0, sem1):
    c = lax.axis_index("c")
    s = lax.axis_index("s")
    wid = s * NC + c
    base = wid * RW

    pltpu.sync_copy(xT_hbm.at[:, pl.ds(base, RW)], xv)

    zeros = jnp.zeros((L,), jnp.float32)
    ones = jnp.ones((L,), jnp.float32)
    lane_row = lax.iota(jnp.int32, L) * VOCAB

    def _z(i, carry):
        for u in range(ZUNROLL):
            acc0[pl.ds(i * (L * ZUNROLL) + u * L, L)] = zeros
            acc1[pl.ds(i * (L * ZUNROLL) + u * L, L)] = zeros
        return carry

    lax.fori_loop(0, C * VOCAB // (L * ZUNROLL), _z, None)

    accs = (acc0, acc1)
    sems = (sem0, sem1)

    def scatter_chunk(acc, k, val):
        for g in range(GPC):
            rowoff = lane_row + g * (L * VOCAB)
            idxs = [
                xv[j, pl.ds(k * C + g * L, L)] + rowoff for j in range(HIST)
            ]
            for idx in idxs:
                if val is None:
                    plsc.addupdate_scatter(acc, [idx], ones)
                else:
                    plsc.store_scatter(acc, [idx], val)

    copies = [None] * NCHUNK
    for k in range(NCHUNK):
        acc = accs[k % 2]
        if k >= 2:
            copies[k - 2].wait()
            scatter_chunk(acc, k - 2, zeros)
        scatter_chunk(acc, k, None)
        copies[k] = pltpu.async_copy(
            acc, out_hbm.at[pl.ds((base + k * C) * VOCAB, C * VOCAB)], sems[k % 2]
        )
    copies[NCHUNK - 2].wait()
    copies[NCHUNK - 1].wait()


def kernel(x):
    xT = x.T
    out = pl.kernel(
        _mh_body,
        out_type=jax.ShapeDtypeStruct((BATCH * VOCAB,), jnp.float32),
        mesh=plsc.VectorSubcoreMesh(core_axis_name="c", subcore_axis_name="s"),
        scratch_types=[
            pltpu.VMEM((HIST, RW), jnp.int32),
            pltpu.VMEM((C * VOCAB,), jnp.float32),
            pltpu.VMEM((C * VOCAB,), jnp.float32),
            pltpu.SemaphoreType.DMA,
            pltpu.SemaphoreType.DMA,
        ],
        compiler_params=pltpu.CompilerParams(needs_layout_passes=False),
    )(xT)
    return out.reshape(BATCH, VOCAB)
